# Initial kernel scaffold; baseline (speedup 1.0000x reference)
import jax
import jax.numpy as jnp
from jax.experimental import pallas as pl


def kernel(x, edge_attr, edge_index, W_emb, b_emb, Wm, bm, P, We, be, Wn, bn, Wp, bp):
    raise NotImplementedError("write your pallas kernel here")


# speedup vs baseline: 3.8830x; 3.8830x over previous
import functools

import jax
import jax.numpy as jnp
from jax import lax
from jax.experimental import pallas as pl
from jax.experimental.pallas import tpu as pltpu
from jax.experimental.pallas import tpu_sc as plsc

N = 10000
E = 320000
NG = 10000
EMB = 128
ED = 16
PD = 64
OD = 64

NC = 2
NS = 16
NW = NC * NS
EPW = E // NW
C = 80
NCH = EPW // C
NPAD = 10240
RPT = NPAD // NS
C2 = 1000
NCH2 = EPW // C2

_f32 = jnp.float32


def _emb_body(x_ref, we_ref, be_ref, wmx_ref, weh_ref, hm_ref, hh_ref):
    h = jnp.dot(x_ref[...], we_ref[...], preferred_element_type=_f32)
    h = h + be_ref[...]
    hm_ref[...] = jnp.dot(h, wmx_ref[...], preferred_element_type=_f32)
    hh_ref[...] = jnp.dot(h, weh_ref[...], preferred_element_type=_f32)


def _run_emb(x, W_emb, b_emb, Wm_x, We_h):
    BN = 400
    grid = (N // BN,)
    return pl.pallas_call(
        _emb_body,
        grid=grid,
        in_specs=[
            pl.BlockSpec((BN, NG), lambda i: (i, 0)),
            pl.BlockSpec((NG, EMB), lambda i: (0, 0)),
            pl.BlockSpec((1, EMB), lambda i: (0, 0)),
            pl.BlockSpec((EMB, PD), lambda i: (0, 0)),
            pl.BlockSpec((EMB, EMB), lambda i: (0, 0)),
        ],
        out_specs=[
            pl.BlockSpec((BN, PD), lambda i: (i, 0)),
            pl.BlockSpec((BN, EMB), lambda i: (i, 0)),
        ],
        out_shape=[
            jax.ShapeDtypeStruct((N, PD), _f32),
            jax.ShapeDtypeStruct((N, EMB), _f32),
        ],
        compiler_params=pltpu.CompilerParams(
            dimension_semantics=("parallel",),
        ),
    )(x, W_emb, b_emb.reshape(1, EMB), Wm_x, We_h)


def _edgefeat_body(ea_ref, src_ref, wme_ref, bm_ref, wpe_ref,
                   em_ref, ep_ref, mx_ref):
    i = pl.program_id(0)
    em_ref[...] = (jnp.dot(ea_ref[...], wme_ref[...],
                           preferred_element_type=_f32) + bm_ref[...])
    ep_ref[...] = jnp.dot(ea_ref[...], wpe_ref[...],
                          preferred_element_type=_f32)

    @pl.when(i == 0)
    def _():
        mx_ref[0, 0] = jnp.int32(0)

    mx_ref[0, 0] = jnp.maximum(mx_ref[0, 0], jnp.max(src_ref[...]))


def _run_edgefeat(edge_attr, src2d, Wm_e, bm, Wp_e):
    BE = 4000
    grid = (E // BE,)
    return pl.pallas_call(
        _edgefeat_body,
        grid=grid,
        in_specs=[
            pl.BlockSpec((BE, ED), lambda i: (i, 0)),
            pl.BlockSpec((1, 1, BE), lambda i: (i, 0, 0)),
            pl.BlockSpec((ED, PD), lambda i: (0, 0)),
            pl.BlockSpec((1, PD), lambda i: (0, 0)),
            pl.BlockSpec((ED, 1), lambda i: (0, 0)),
        ],
        out_specs=[
            pl.BlockSpec((BE, PD), lambda i: (i, 0)),
            pl.BlockSpec((BE, 1), lambda i: (i, 0)),
            pl.BlockSpec((1, 1), lambda i: (0, 0),
                         memory_space=pltpu.SMEM),
        ],
        out_shape=[
            jax.ShapeDtypeStruct((E, PD), _f32),
            jax.ShapeDtypeStruct((E, 1), _f32),
            jax.ShapeDtypeStruct((1, 1), jnp.int32),
        ],
        compiler_params=pltpu.CompilerParams(
            dimension_semantics=("arbitrary",),
        ),
    )(edge_attr, src2d, Wm_e, bm.reshape(1, PD), Wp_e)


def _aggr_body(hm_hbm, em_hbm, src_hbm, dst_hbm, off_hbm, p_hbm, out_hbm,
               acc_sh, src_v, dst_v, pidx_v, off_v,
               hm_v, em_v, prow_v, msg_v, sem1, sem2, sem3):
    c = lax.axis_index("c")
    s = lax.axis_index("s")
    wid = s * NC + c

    def zrow(r, _):
        for g in range(PD // 16):
            msg_v[r, pl.ds(g * 16, 16)] = jnp.zeros((16,), _f32)
        return 0

    lax.fori_loop(0, C, zrow, 0)

    def zcopy(j, _):
        pltpu.sync_copy(msg_v, acc_sh.at[pl.ds(s * RPT + j * C, C)])
        return 0

    lax.fori_loop(0, RPT // C, zcopy, 0)
    plsc.subcore_barrier()

    pltpu.sync_copy(off_hbm, off_v)
    offv = off_v[...]
    ebase = wid * EPW

    def chunk(i, _):
        base = ebase + i * C
        pltpu.sync_copy(src_hbm.at[pl.ds(base, C)], src_v)
        pltpu.sync_copy(dst_hbm.at[pl.ds(base, C)], dst_v)
        cp_em = pltpu.async_copy(em_hbm.at[pl.ds(base, C)], em_v, sem2)
        cp_hm = pltpu.async_copy(hm_hbm.at[src_v], hm_v, sem1)
        for g in range(C // 16):
            pidx_v[pl.ds(g * 16, 16)] = src_v[pl.ds(g * 16, 16)] - offv
        cp_p = pltpu.async_copy(p_hbm.at[pidx_v], prow_v, sem3)
        cp_em.wait()
        cp_hm.wait()
        cp_p.wait()

        def row(r, _):
            for g in range(PD // 16):
                m = hm_v[r, pl.ds(g * 16, 16)] + em_v[r, pl.ds(g * 16, 16)]
                m = jnp.maximum(m, 0.0) * prow_v[r, pl.ds(g * 16, 16)]
                msg_v[r, pl.ds(g * 16, 16)] = m
            return 0

        lax.fori_loop(0, C, row, 0)
        pltpu.sync_copy(msg_v, acc_sh.at[dst_v], add=True)
        return 0

    lax.fori_loop(0, NCH, chunk, 0)
    plsc.subcore_barrier()
    pltpu.sync_copy(acc_sh.at[pl.ds(s * RPT, RPT)],
                    out_hbm.at[c, pl.ds(s * RPT, RPT)])


def _run_aggr(hm, em, src, dst, off16, P):
    mesh = plsc.VectorSubcoreMesh(core_axis_name="c", subcore_axis_name="s")
    k = functools.partial(
        pl.kernel,
        mesh=mesh,
        out_type=jax.ShapeDtypeStruct((NC, NPAD, PD), _f32),
        scratch_types=[
            pltpu.VMEM_SHARED((NPAD, PD), _f32),
            pltpu.VMEM((C,), jnp.int32),
            pltpu.VMEM((C,), jnp.int32),
            pltpu.VMEM((C,), jnp.int32),
            pltpu.VMEM((16,), jnp.int32),
            pltpu.VMEM((C, PD), _f32),
            pltpu.VMEM((C, PD), _f32),
            pltpu.VMEM((C, PD), _f32),
            pltpu.VMEM((C, PD), _f32),
            pltpu.SemaphoreType.DMA,
            pltpu.SemaphoreType.DMA,
            pltpu.SemaphoreType.DMA,
        ],
        compiler_params=pltpu.CompilerParams(use_tc_tiling_on_sc=False),
    )(_aggr_body)
    return k(hm, em, src, dst, off16, P)


def _update_body(parts_ref, hh_ref, wea_ref, be_ref, wn_ref, bn_ref, wp_ref,
                 np_ref, hp_ref):
    aggr = parts_ref[0] + parts_ref[1]
    h2 = jnp.dot(aggr, wea_ref[...], preferred_element_type=_f32)
    h2 = jnp.maximum(h2 + hh_ref[...] + be_ref[...], 0.0)
    np_ref[...] = jnp.dot(h2, wn_ref[...], preferred_element_type=_f32) + bn_ref[...]
    hp_ref[...] = jnp.dot(h2, wp_ref[...], preferred_element_type=_f32)


def _run_update(parts, hh, We_a, be, Wn, bn, Wp12):
    BN = 400
    grid = (N // BN,)
    return pl.pallas_call(
        _update_body,
        grid=grid,
        in_specs=[
            pl.BlockSpec((NC, BN, PD), lambda i: (0, i, 0)),
            pl.BlockSpec((BN, EMB), lambda i: (i, 0)),
            pl.BlockSpec((PD, EMB), lambda i: (0, 0)),
            pl.BlockSpec((1, EMB), lambda i: (0, 0)),
            pl.BlockSpec((EMB, OD), lambda i: (0, 0)),
            pl.BlockSpec((1, OD), lambda i: (0, 0)),
            pl.BlockSpec((EMB, 2), lambda i: (0, 0)),
        ],
        out_specs=[
            pl.BlockSpec((BN, OD), lambda i: (i, 0)),
            pl.BlockSpec((BN, 2), lambda i: (i, 0)),
        ],
        out_shape=[
            jax.ShapeDtypeStruct((N, OD), _f32),
            jax.ShapeDtypeStruct((N, 2), _f32),
        ],
        compiler_params=pltpu.CompilerParams(
            dimension_semantics=("parallel",),
        ),
    )(parts, hh, We_a, be.reshape(1, EMB), Wn, bn.reshape(1, OD), Wp12)


def _epred_body(hp1_hbm, hp2_hbm, src_hbm, dst_hbm, ep_hbm, bp_hbm, out_hbm,
                src_v, dst_v, ep_v, a_v, b_v, o_v, bp_v, sem1, sem2, sem3):
    c = lax.axis_index("c")
    s = lax.axis_index("s")
    wid = s * NC + c
    pltpu.sync_copy(bp_hbm, bp_v)
    bpv = bp_v[...]
    ebase = wid * EPW

    def chunk(i, _):
        base = ebase + i * C
        pltpu.sync_copy(src_hbm.at[pl.ds(base, C)], src_v)
        pltpu.sync_copy(dst_hbm.at[pl.ds(base, C)], dst_v)
        cp_e = pltpu.async_copy(ep_hbm.at[pl.ds(base, C)], ep_v, sem3)
        cp_a = pltpu.async_copy(hp1_hbm.at[src_v], a_v, sem1)
        cp_b = pltpu.async_copy(hp2_hbm.at[dst_v], b_v, sem2)
        cp_e.wait()
        cp_a.wait()
        cp_b.wait()
        for g in range(C // 16):
            o_v[pl.ds(g * 16, 16)] = (a_v[pl.ds(g * 16, 16)]
                                      + b_v[pl.ds(g * 16, 16)]
                                      + ep_v[pl.ds(g * 16, 16)] + bpv)
        pltpu.sync_copy(o_v, out_hbm.at[pl.ds(base, C)])
        return 0

    lax.fori_loop(0, NCH, chunk, 0)


def _run_epred(hp1, hp2, src, dst, ep, bp16):
    mesh = plsc.VectorSubcoreMesh(core_axis_name="c", subcore_axis_name="s")
    k = functools.partial(
        pl.kernel,
        mesh=mesh,
        out_type=jax.ShapeDtypeStruct((E,), _f32),
        scratch_types=[
            pltpu.VMEM((C,), jnp.int32),
            pltpu.VMEM((C,), jnp.int32),
            pltpu.VMEM((C,), _f32),
            pltpu.VMEM((C,), _f32),
            pltpu.VMEM((C,), _f32),
            pltpu.VMEM((C,), _f32),
            pltpu.VMEM((16,), _f32),
            pltpu.SemaphoreType.DMA,
            pltpu.SemaphoreType.DMA,
            pltpu.SemaphoreType.DMA,
        ],
        compiler_params=pltpu.CompilerParams(use_tc_tiling_on_sc=False),
    )(_epred_body)
    return k(hp1, hp2, src, dst, ep, bp16)


def kernel(x, edge_attr, edge_index, W_emb, b_emb, Wm, bm, P, We, be,
           Wn, bn, Wp, bp):
    src = edge_index[0]
    dst = edge_index[1]
    Wm_x, Wm_e = Wm[:EMB], Wm[EMB:]
    We_a, We_h = We[:PD], We[PD:]
    Wp12 = jnp.concatenate([Wp[:EMB], Wp[EMB:2 * EMB]], axis=1)
    Wp_e = Wp[2 * EMB:]

    em, ep, mx = _run_edgefeat(edge_attr, src.reshape(E // 4000, 1, 4000),
                               Wm_e, bm, Wp_e)
    hm, hh = _run_emb(x, W_emb, b_emb, Wm_x, We_h)

    off16 = jnp.broadcast_to(mx[0, 0] - jnp.int32(NG - 1), (16,))
    parts = _run_aggr(hm, em, src, dst, off16, P)

    node_pred, hp = _run_update(parts, hh, We_a, be, Wn, bn, Wp12)

    bp16 = jnp.broadcast_to(bp[0], (16,))
    edge_pred = _run_epred(hp[:, 0], hp[:, 1], src, dst, ep[:, 0], bp16)
    return node_pred, edge_pred[:, None]
